# baseline (speedup 1.0000x reference)
import jax
import jax.numpy as jnp
from jax import lax
from jax.experimental import pallas as pl
from jax.experimental.pallas import tpu as pltpu

B, T, CIN = 8, 2048, 1
H, L, E, K, NH, F, PL_ = 1024, 2, 8, 2, 16, 1024, 32
NP = T // PL_
N = B * NP
CAP = int(1.25 * N * K / E)
DH = H // NH
HZ = [1, 8, 32, 64]


_HP = None


def _dot(a, b, dn=None):
    if dn is None:
        return jnp.dot(a, b, precision=_HP)
    return lax.dot_general(a, b, dn, precision=_HP)


def _rms(x, w):
    return x * w * lax.rsqrt(jnp.mean(x * x, axis=-1, keepdims=True) + 1e-6)


def _silu(x):
    return x * (1.0 / (1.0 + jnp.exp(-x)))


def _mega_body(p_ref, embW_ref, gateW_ref, ln1_ref, ln2_ref, wr_ref,
               wh_ref, bh_ref,
               qkvW_hbm, oW_hbm, w1_hbm, w2_hbm,
               outs_ref, stats_ref,
               qkvbuf, obuf, w1buf, w2buf,
               sem_qkv, sem_o, sem_w1, sem_w2):

    def qkv_copy(l):
        return pltpu.make_async_copy(qkvW_hbm.at[l], qkvbuf, sem_qkv)

    def o_copy(l):
        return pltpu.make_async_copy(oW_hbm.at[l], obuf, sem_o)

    def w1_copy(l, e):
        return pltpu.make_async_copy(w1_hbm.at[l, e], w1buf.at[e % 3],
                                     sem_w1.at[e % 3])

    def w2_copy(l, e):
        return pltpu.make_async_copy(w2_hbm.at[l, e], w2buf.at[e % 3],
                                     sem_w2.at[e % 3])

    qkv_copy(0).start()
    o_copy(0).start()
    w1_copy(0, 0).start()
    w2_copy(0, 0).start()
    w1_copy(0, 1).start()
    w2_copy(0, 1).start()

    p = p_ref[...]
    h = _silu(_dot(p, gateW_ref[...])) * _dot(p, embW_ref[...])

    r_iota = lax.broadcasted_iota(jnp.int32, (N, N), 0)
    c_iota = lax.broadcasted_iota(jnp.int32, (N, N), 1)
    mask = ((r_iota // NP) == (c_iota // NP)) & (r_iota >= c_iota)
    lt = (r_iota > c_iota).astype(jnp.float32)
    e_iota = lax.broadcasted_iota(jnp.int32, (N, E), 1)
    c_iota2 = lax.broadcasted_iota(jnp.int32, (N, CAP), 1)

    stats = jnp.zeros((1, E), jnp.float32)

    for l in range(L):
        qkv_copy(l).wait()
        ni = _rms(h, ln1_ref[l:l + 1])
        qkv = _dot(ni, qkvbuf[...])
        ctxs = []
        for hd in range(NH):
            q = qkv[:, hd * DH:(hd + 1) * DH]
            k = qkv[:, H + hd * DH:H + (hd + 1) * DH]
            v = qkv[:, 2 * H + hd * DH:2 * H + (hd + 1) * DH]
            s = _dot(q, k, (((1,), (1,)), ((), ())))
            s = jnp.where(mask, s * (1.0 / (DH ** 0.5)), -1e9)
            s = s - jnp.max(s, axis=-1, keepdims=True)
            pe = jnp.exp(s)
            a = pe / jnp.sum(pe, axis=-1, keepdims=True)
            ctxs.append(_dot(a, v))
        ctx = jnp.concatenate(ctxs, axis=-1)
        o_copy(l).wait()
        attn = _dot(ctx, obuf[...])
        hs = _rms(attn + ni, ln2_ref[l:l + 1])

        logits = _dot(hs, wr_ref[l])
        m = jnp.max(logits, axis=-1, keepdims=True)
        ex = jnp.exp(logits - m)
        probs = ex / jnp.sum(ex, axis=-1, keepdims=True)
        m0 = jnp.max(probs, axis=-1, keepdims=True)
        i0 = jnp.min(jnp.where(probs == m0, e_iota, E), axis=-1, keepdims=True)
        oh0 = (e_iota == i0).astype(jnp.float32)
        probs1 = jnp.where(e_iota == i0, -1.0, probs)
        m1 = jnp.max(probs1, axis=-1, keepdims=True)
        i1 = jnp.min(jnp.where(probs1 == m1, e_iota, E), axis=-1, keepdims=True)
        oh1 = (e_iota == i1).astype(jnp.float32)
        denom = m0 + m1 + 1e-9
        g0 = m0 / denom
        g1 = m1 / denom
        cum0 = jnp.dot(lt, oh0)
        pos0 = jnp.sum(cum0 * oh0, axis=-1, keepdims=True)
        total0 = jnp.sum(oh0, axis=0, keepdims=True)
        cum1 = jnp.dot(lt, oh1) + total0
        pos1 = jnp.sum(cum1 * oh1, axis=-1, keepdims=True)
        keep0 = (pos0 < CAP).astype(jnp.float32)
        keep1 = (pos1 < CAP).astype(jnp.float32)
        stats = stats + jnp.sum(keep0 * oh0 + keep1 * oh1, axis=0,
                                keepdims=True)
        kg0 = keep0 * g0
        kg1 = keep1 * g1
        pos0i = pos0.astype(jnp.int32)
        pos1i = pos1.astype(jnp.int32)

        y = None
        for e in range(E):
            if e + 2 < E:
                w1_copy(l, e + 2).start()
                w2_copy(l, e + 2).start()
            elif e + 2 == E and l + 1 < L:
                qkv_copy(l + 1).start()
                o_copy(l + 1).start()
            w1_copy(l, e).wait()
            w2_copy(l, e).wait()
            d0 = ((i0 == e) & (pos0i == c_iota2)).astype(jnp.float32)
            d1 = ((i1 == e) & (pos1i == c_iota2)).astype(jnp.float32)
            disp = d0 + d1
            x_e = _dot(disp, hs, (((0,), (0,)), ((), ())))
            hmid = _silu(_dot(x_e, w1buf[e % 3]))
            eout = _dot(hmid, w2buf[e % 3])
            comb = kg0 * d0 + kg1 * d1
            ye = _dot(comb, eout)
            y = ye if y is None else y + ye
        if l + 1 < L:
            w1_copy(l + 1, 0).start()
            w2_copy(l + 1, 0).start()
            w1_copy(l + 1, 1).start()
            w2_copy(l + 1, 1).start()
        h = 2.0 * hs + y

    outs_ref[...] = _dot(h, wh_ref[...]) + bh_ref[...]
    stats_ref[...] = stats


def kernel(x, emb_W, gate_W, ln1_w, qkv_W, o_W, ln2_w, router_W,
           exp_W1, exp_W2, hW1, hb1, hW8, hb8, hW32, hb32, hW64, hb64):
    f32 = jnp.float32
    p = x.reshape(B, NP, PL_, CIN).transpose(0, 1, 3, 2).reshape(N, CIN * PL_)
    Wh = jnp.concatenate([hW1, hW8, hW32, hW64], axis=1)
    bh = jnp.concatenate([hb1, hb8, hb32, hb64]).reshape(1, -1)

    vmem = pl.BlockSpec(memory_space=pltpu.MemorySpace.HBM)
    outs, stats = pl.pallas_call(
        _mega_body,
        in_specs=[pl.BlockSpec((N, CIN * PL_), lambda: (0, 0)),
                  pl.BlockSpec((CIN * PL_, H), lambda: (0, 0)),
                  pl.BlockSpec((CIN * PL_, H), lambda: (0, 0)),
                  pl.BlockSpec((L, H), lambda: (0, 0)),
                  pl.BlockSpec((L, H), lambda: (0, 0)),
                  pl.BlockSpec((L, H, E), lambda: (0, 0, 0)),
                  pl.BlockSpec((H, sum(HZ)), lambda: (0, 0)),
                  pl.BlockSpec((1, sum(HZ)), lambda: (0, 0)),
                  vmem, vmem, vmem, vmem],
        out_shape=(jax.ShapeDtypeStruct((N, sum(HZ)), f32),
                   jax.ShapeDtypeStruct((1, E), f32)),
        scratch_shapes=[
            pltpu.VMEM((H, 3 * H), f32),
            pltpu.VMEM((H, H), f32),
            pltpu.VMEM((3, H, F), f32),
            pltpu.VMEM((3, F, H), f32),
            pltpu.SemaphoreType.DMA,
            pltpu.SemaphoreType.DMA,
            pltpu.SemaphoreType.DMA((3,)),
            pltpu.SemaphoreType.DMA((3,)),
        ],
        compiler_params=pltpu.CompilerParams(
            vmem_limit_bytes=100 * 1024 * 1024),
    )(p, emb_W, gate_W, ln1_w, ln2_w, router_W, Wh, bh,
      qkv_W, o_W, exp_W1, exp_W2)

    res, off = [], 0
    for z in HZ:
        res.append(outs[:, off:off + z].reshape(B, NP, z))
        off += z
    return (*res, stats[0])


# speedup vs baseline: 1.0068x; 1.0068x over previous
import jax
import jax.numpy as jnp
from jax import lax
from jax.experimental import pallas as pl
from jax.experimental.pallas import tpu as pltpu

B, T, CIN = 8, 2048, 1
H, L, E, K, NH, F, PL_ = 1024, 2, 8, 2, 16, 1024, 32
NP = T // PL_
N = B * NP
CAP = int(1.25 * N * K / E)
DH = H // NH
HZ = [1, 8, 32, 64]


_HP = None


def _dot(a, b, dn=None):
    if dn is None:
        return jnp.dot(a, b, precision=_HP)
    return lax.dot_general(a, b, dn, precision=_HP)


def _rms(x, w):
    return x * w * lax.rsqrt(jnp.mean(x * x, axis=-1, keepdims=True) + 1e-6)


def _silu(x):
    return x * (1.0 / (1.0 + jnp.exp(-x)))


def _mega_body(p_ref, embW_ref, gateW_ref, ln1_ref, ln2_ref, wr_ref,
               hw1_ref, hb1_ref, hw8_ref, hb8_ref, hw32_ref, hb32_ref,
               hw64_ref, hb64_ref,
               qkvW_hbm, oW_hbm, w1_hbm, w2_hbm,
               o1_ref, o8_ref, o32_ref, o64_ref, stats_ref,
               qkvbuf, obuf, w1buf, w2buf,
               sem_qkv, sem_o, sem_w1, sem_w2):

    def qkv_copy(l):
        return pltpu.make_async_copy(qkvW_hbm.at[l], qkvbuf, sem_qkv)

    def o_copy(l):
        return pltpu.make_async_copy(oW_hbm.at[l], obuf, sem_o)

    def w1_copy(l, e):
        return pltpu.make_async_copy(w1_hbm.at[l, e], w1buf.at[e % 3],
                                     sem_w1.at[e % 3])

    def w2_copy(l, e):
        return pltpu.make_async_copy(w2_hbm.at[l, e], w2buf.at[e % 3],
                                     sem_w2.at[e % 3])

    qkv_copy(0).start()
    o_copy(0).start()
    w1_copy(0, 0).start()
    w2_copy(0, 0).start()
    w1_copy(0, 1).start()
    w2_copy(0, 1).start()

    p = p_ref[...]
    h = _silu(_dot(p, gateW_ref[...])) * _dot(p, embW_ref[...])

    r_iota = lax.broadcasted_iota(jnp.int32, (N, N), 0)
    c_iota = lax.broadcasted_iota(jnp.int32, (N, N), 1)
    mask = ((r_iota // NP) == (c_iota // NP)) & (r_iota >= c_iota)
    lt = (r_iota > c_iota).astype(jnp.float32)
    e_iota = lax.broadcasted_iota(jnp.int32, (N, E), 1)
    c_iota2 = lax.broadcasted_iota(jnp.int32, (N, CAP), 1)

    stats = jnp.zeros((1, E), jnp.float32)

    for l in range(L):
        qkv_copy(l).wait()
        ni = _rms(h, ln1_ref[l:l + 1])
        qkv = _dot(ni, qkvbuf[...])
        ctxs = []
        for hd in range(NH):
            q = qkv[:, hd * DH:(hd + 1) * DH]
            k = qkv[:, H + hd * DH:H + (hd + 1) * DH]
            v = qkv[:, 2 * H + hd * DH:2 * H + (hd + 1) * DH]
            s = _dot(q, k, (((1,), (1,)), ((), ())))
            s = jnp.where(mask, s * (1.0 / (DH ** 0.5)), -1e9)
            s = s - jnp.max(s, axis=-1, keepdims=True)
            pe = jnp.exp(s)
            a = pe / jnp.sum(pe, axis=-1, keepdims=True)
            ctxs.append(_dot(a, v))
        ctx = jnp.concatenate(ctxs, axis=-1)
        o_copy(l).wait()
        attn = _dot(ctx, obuf[...])
        hs = _rms(attn + ni, ln2_ref[l:l + 1])

        logits = _dot(hs, wr_ref[l])
        m = jnp.max(logits, axis=-1, keepdims=True)
        ex = jnp.exp(logits - m)
        probs = ex / jnp.sum(ex, axis=-1, keepdims=True)
        m0 = jnp.max(probs, axis=-1, keepdims=True)
        i0 = jnp.min(jnp.where(probs == m0, e_iota, E), axis=-1, keepdims=True)
        oh0 = (e_iota == i0).astype(jnp.float32)
        probs1 = jnp.where(e_iota == i0, -1.0, probs)
        m1 = jnp.max(probs1, axis=-1, keepdims=True)
        i1 = jnp.min(jnp.where(probs1 == m1, e_iota, E), axis=-1, keepdims=True)
        oh1 = (e_iota == i1).astype(jnp.float32)
        denom = m0 + m1 + 1e-9
        g0 = m0 / denom
        g1 = m1 / denom
        cum0 = jnp.dot(lt, oh0)
        pos0 = jnp.sum(cum0 * oh0, axis=-1, keepdims=True)
        total0 = jnp.sum(oh0, axis=0, keepdims=True)
        cum1 = jnp.dot(lt, oh1) + total0
        pos1 = jnp.sum(cum1 * oh1, axis=-1, keepdims=True)
        keep0 = (pos0 < CAP).astype(jnp.float32)
        keep1 = (pos1 < CAP).astype(jnp.float32)
        stats = stats + jnp.sum(keep0 * oh0 + keep1 * oh1, axis=0,
                                keepdims=True)
        kg0 = keep0 * g0
        kg1 = keep1 * g1
        pos0i = pos0.astype(jnp.int32)
        pos1i = pos1.astype(jnp.int32)

        y = None
        for e in range(E):
            if e + 2 < E:
                w1_copy(l, e + 2).start()
                w2_copy(l, e + 2).start()
            elif e + 2 == E and l + 1 < L:
                qkv_copy(l + 1).start()
                o_copy(l + 1).start()
            w1_copy(l, e).wait()
            w2_copy(l, e).wait()
            d0 = ((i0 == e) & (pos0i == c_iota2)).astype(jnp.float32)
            d1 = ((i1 == e) & (pos1i == c_iota2)).astype(jnp.float32)
            disp = d0 + d1
            x_e = _dot(disp, hs, (((0,), (0,)), ((), ())))
            hmid = _silu(_dot(x_e, w1buf[e % 3]))
            eout = _dot(hmid, w2buf[e % 3])
            comb = kg0 * d0 + kg1 * d1
            ye = _dot(comb, eout)
            y = ye if y is None else y + ye
        if l + 1 < L:
            w1_copy(l + 1, 0).start()
            w2_copy(l + 1, 0).start()
            w1_copy(l + 1, 1).start()
            w2_copy(l + 1, 1).start()
        h = 2.0 * hs + y

    o1_ref[...] = (_dot(h, hw1_ref[...]) + hb1_ref[...]).reshape(B, NP, 1)
    o8_ref[...] = (_dot(h, hw8_ref[...]) + hb8_ref[...]).reshape(B, NP, 8)
    o32_ref[...] = (_dot(h, hw32_ref[...]) + hb32_ref[...]).reshape(B, NP, 32)
    o64_ref[...] = (_dot(h, hw64_ref[...]) + hb64_ref[...]).reshape(B, NP, 64)
    stats_ref[...] = stats.reshape(E)


def kernel(x, emb_W, gate_W, ln1_w, qkv_W, o_W, ln2_w, router_W,
           exp_W1, exp_W2, hW1, hb1, hW8, hb8, hW32, hb32, hW64, hb64):
    f32 = jnp.float32
    p = x.reshape(N, CIN * PL_)

    hbm = pl.BlockSpec(memory_space=pltpu.MemorySpace.HBM)
    o1, o8, o32, o64, stats = pl.pallas_call(
        _mega_body,
        in_specs=[pl.BlockSpec((N, CIN * PL_), lambda: (0, 0)),
                  pl.BlockSpec((CIN * PL_, H), lambda: (0, 0)),
                  pl.BlockSpec((CIN * PL_, H), lambda: (0, 0)),
                  pl.BlockSpec((L, H), lambda: (0, 0)),
                  pl.BlockSpec((L, H), lambda: (0, 0)),
                  pl.BlockSpec((L, H, E), lambda: (0, 0, 0)),
                  pl.BlockSpec((H, 1), lambda: (0, 0)),
                  pl.BlockSpec((1,), lambda: (0,)),
                  pl.BlockSpec((H, 8), lambda: (0, 0)),
                  pl.BlockSpec((8,), lambda: (0,)),
                  pl.BlockSpec((H, 32), lambda: (0, 0)),
                  pl.BlockSpec((32,), lambda: (0,)),
                  pl.BlockSpec((H, 64), lambda: (0, 0)),
                  pl.BlockSpec((64,), lambda: (0,)),
                  hbm, hbm, hbm, hbm],
        out_shape=(jax.ShapeDtypeStruct((B, NP, 1), f32),
                   jax.ShapeDtypeStruct((B, NP, 8), f32),
                   jax.ShapeDtypeStruct((B, NP, 32), f32),
                   jax.ShapeDtypeStruct((B, NP, 64), f32),
                   jax.ShapeDtypeStruct((E,), f32)),
        scratch_shapes=[
            pltpu.VMEM((H, 3 * H), f32),
            pltpu.VMEM((H, H), f32),
            pltpu.VMEM((3, H, F), f32),
            pltpu.VMEM((3, F, H), f32),
            pltpu.SemaphoreType.DMA,
            pltpu.SemaphoreType.DMA,
            pltpu.SemaphoreType.DMA((3,)),
            pltpu.SemaphoreType.DMA((3,)),
        ],
        compiler_params=pltpu.CompilerParams(
            vmem_limit_bytes=100 * 1024 * 1024),
    )(p, emb_W, gate_W, ln1_w, ln2_w, router_W,
      hW1, hb1, hW8, hb8, hW32, hb32, hW64, hb64,
      qkv_W, o_W, exp_W1, exp_W2)
    return (o1, o8, o32, o64, stats)
